# baseline (speedup 1.0000x reference)
import functools
import jax
import jax.numpy as jnp
from jax import lax
from jax.experimental import pallas as pl
from jax.experimental.pallas import tpu as pltpu

NUM_GRAPHS = 64
_EPS = 1e-5
_HI = lax.Precision.HIGHEST


def _seg(Mt, v):
    return lax.dot_general(Mt, v, (((1,), (0,)), ((), ())), precision=_HI)


def _bcast(Mt, v):
    return lax.dot_general(Mt, v, (((0,), (0,)), ((), ())), precision=_HI)


def _graph_norm(x, Mt, inv_cnt, w, b, ms):
    mean = _seg(Mt, x) * inv_cnt
    t = x - _bcast(Mt, mean) * ms
    var = _seg(Mt, t * t) * inv_cnt
    std = jnp.sqrt(var + _EPS)
    return w * t / _bcast(Mt, std) + b


def _one_hot_t(batch1n, g):
    gi = lax.broadcasted_iota(jnp.int32, (g, 1), 0)
    return (batch1n == gi).astype(jnp.float32)


def _inv_cnt(Mt, n):
    ones = jnp.ones((n, 1), jnp.float32)
    cnt = lax.dot_general(Mt, ones, (((1,), (0,)), ((), ())), precision=_HI)
    return 1.0 / jnp.maximum(cnt, 1.0)


def _k1_body(x_ref, batch_ref, w_ref, b_ref, ms_ref, Wr_ref, bl_ref,
             h_ref, r_ref):
    x = x_ref[...]
    Mt = _one_hot_t(batch_ref[...], NUM_GRAPHS)
    h = _graph_norm(x, Mt, _inv_cnt(Mt, x.shape[0]),
                    w_ref[...], b_ref[...], ms_ref[...])
    h_ref[...] = h
    r_ref[...] = x + jnp.dot(h, Wr_ref[...],
                             preferred_element_type=jnp.float32) + bl_ref[...]


def _scatter_body(src_ref, dst_ref, h_ref, acc_ref, *, chunk):
    @pl.when(pl.program_id(0) == 0)
    def _init():
        acc_ref[...] = jnp.full_like(acc_ref, -jnp.inf)

    def body(i, _):
        s = src_ref[i]
        d = dst_ref[i]
        row = jnp.maximum(acc_ref[pl.ds(d, 1), :], h_ref[pl.ds(s, 1), :])
        acc_ref[pl.ds(d, 1), :] = row
        return 0

    lax.fori_loop(0, chunk, body, 0)


def _k3_body(agg_ref, r_ref, Wl_ref, batch_ref, w_ref, b_ref, ms_ref,
             out_ref):
    agg = agg_ref[...]
    agg = jnp.where(jnp.isfinite(agg), agg, 0.0)
    pre = jnp.dot(agg, Wl_ref[...], preferred_element_type=jnp.float32) \
        + r_ref[...]
    h2 = jnp.maximum(pre, 0.0)
    Mt = _one_hot_t(batch_ref[...], NUM_GRAPHS)
    out_ref[...] = _graph_norm(h2, Mt, _inv_cnt(Mt, h2.shape[0]),
                               w_ref[...], b_ref[...], ms_ref[...])


def kernel(x, edge_index, batch, W_l, b_l, W_r,
           gn1_w, gn1_b, gn1_ms, gn2_w, gn2_b, gn2_ms):
    n, d = x.shape
    e = edge_index.shape[1]
    batch1n = batch.reshape(1, n)
    row = lambda v: v.reshape(1, d)

    def full(shape):
        return pl.BlockSpec(shape, lambda *_: tuple(0 for _ in shape))

    h, r = pl.pallas_call(
        _k1_body,
        out_shape=(jax.ShapeDtypeStruct((n, d), jnp.float32),
                   jax.ShapeDtypeStruct((n, d), jnp.float32)),
    )(x, batch1n, row(gn1_w), row(gn1_b), row(gn1_ms), W_r, row(b_l))

    chunk = 512
    while e % chunk:
        chunk //= 2
    steps = e // chunk
    agg = pl.pallas_call(
        functools.partial(_scatter_body, chunk=chunk),
        grid=(steps,),
        in_specs=[
            pl.BlockSpec((chunk,), lambda i: (i,), memory_space=pltpu.SMEM),
            pl.BlockSpec((chunk,), lambda i: (i,), memory_space=pltpu.SMEM),
            full((n, d)),
        ],
        out_specs=full((n, d)),
        out_shape=jax.ShapeDtypeStruct((n, d), jnp.float32),
    )(edge_index[0], edge_index[1], h)

    out = pl.pallas_call(
        _k3_body,
        out_shape=jax.ShapeDtypeStruct((n, d), jnp.float32),
    )(agg, r, W_l, batch1n, row(gn2_w), row(gn2_b), row(gn2_ms))
    return out
